# baseline (speedup 1.0000x reference)
import functools

import jax
import jax.numpy as jnp
import numpy as np
from jax import lax
from jax.experimental import pallas as pl
from jax.experimental.pallas import tpu as pltpu
from jax.experimental.pallas import tpu_sc as plsc


def _mlp_select_body(P, T, HEAD, shifts, pidx_ref, x_ref, w1_ref, b1_ref,
                     w2_ref, b2_ref, out_ref):
    x = x_ref[...]
    h = jnp.tanh(
        jax.lax.dot(x, w1_ref[...], precision=jax.lax.Precision.HIGHEST)
        + b1_ref[...])
    y = (jax.lax.dot(h, w2_ref[...], precision=jax.lax.Precision.HIGHEST)
         + b2_ref[...])
    out_ref[...] = jnp.zeros(out_ref.shape, out_ref.dtype)
    slices = [lax.slice(y, (q * T, 0), ((q + 1) * T, y.shape[1]))
              for q in range(P)]
    B = out_ref.shape[0] // HEAD
    for b in range(B):
        p = pidx_ref[b]
        sel = slices[0]
        for q in range(1, P):
            sel = jnp.where(p == q, slices[q], sel)
        out_ref[pl.ds(b * HEAD + shifts[b], T), :] = sel


def _prefix_prompts(prefix_indices, input_tokens, W1, b1, W2, b2, HEAD,
                    shifts):
    P, T, MID = input_tokens.shape
    H = W1.shape[1]
    B = prefix_indices.shape[0]
    x = input_tokens.reshape(P * T, MID)
    return pl.pallas_call(
        functools.partial(_mlp_select_body, P, T, HEAD, shifts),
        out_shape=jax.ShapeDtypeStruct((B * HEAD, H), jnp.float32),
        in_specs=[
            pl.BlockSpec(memory_space=pltpu.SMEM),
            pl.BlockSpec(memory_space=pltpu.VMEM),
            pl.BlockSpec(memory_space=pltpu.VMEM),
            pl.BlockSpec(memory_space=pltpu.VMEM),
            pl.BlockSpec(memory_space=pltpu.VMEM),
            pl.BlockSpec(memory_space=pltpu.VMEM),
        ],
        out_specs=pl.BlockSpec(memory_space=pltpu.VMEM),
    )(prefix_indices, x, W1, b1.reshape(1, H), W2, b2.reshape(1, H))


def _row_source(grow, S, T):
    ROWS = T + S
    b = grow // ROWS
    off = grow % ROWS
    if off < T:
        return None
    return b * S + (off - T)


def _sc_gather(wte_weight, tok_flat, pos_map, prompts, B, S, T, H,
               NC, NW, WPB, HEAD, IW, CH, NCH, LAST, ROWS):
    LANES = 16
    JR = HEAD

    mesh = plsc.VectorSubcoreMesh(core_axis_name="c", subcore_axis_name="s")

    @functools.partial(
        pl.kernel,
        out_type=jax.ShapeDtypeStruct((B * ROWS, H), jnp.float32),
        mesh=mesh,
        scratch_types=[
            pltpu.VMEM((IW,), jnp.int32),
            pltpu.VMEM((IW,), jnp.int32),
            pltpu.VMEM((2, CH, H), jnp.float32),
            pltpu.VMEM((JR, H), jnp.float32),
            pltpu.VMEM((JR, H), jnp.float32),
            pltpu.SemaphoreType.DMA,
            pltpu.SemaphoreType.DMA,
            pltpu.SemaphoreType.DMA,
            pltpu.SemaphoreType.DMA,
            pltpu.SemaphoreType.DMA,
        ],
    )
    def k(wte_hbm, tok_hbm, map_hbm, prompts_hbm, out_hbm,
          map_v, idx_v, rows_v, pref_v, joint_v, sg0, sg1, so0, so1, sh):
        wid = lax.axis_index("s") * NC + lax.axis_index("c")
        b = wid // WPB
        sub = wid % WPB
        gb = ((ROWS * b) // 8) * 8
        m = ROWS * b - gb
        rb = gb + JR

        pltpu.sync_copy(map_hbm.at[pl.ds(wid * IW, IW)], map_v)
        e0 = pltpu.async_copy(
            tok_hbm.at[map_v.at[pl.ds(0, 96)]], idx_v.at[pl.ds(0, 96)], sg0)
        e1 = pltpu.async_copy(
            tok_hbm.at[map_v.at[pl.ds(96, 96)]], idx_v.at[pl.ds(96, 96)],
            sg1)
        e2 = pltpu.async_copy(
            tok_hbm.at[map_v.at[pl.ds(192, IW - 192)]],
            idx_v.at[pl.ds(192, IW - 192)], so0)
        e0.wait()
        e1.wait()
        e2.wait()

        sg = (sg0, sg1)
        so = (so0, so1)
        gathers = [None, None]
        writes = [None, None]
        gathers[0] = pltpu.async_copy(
            wte_hbm.at[idx_v.at[pl.ds(JR, CH)]], rows_v.at[0], sg[0])

        @pl.when(sub == 0)
        def _():
            pltpu.sync_copy(prompts_hbm.at[pl.ds(b * HEAD, HEAD)], pref_v)
            pltpu.async_copy(
                wte_hbm.at[idx_v.at[pl.ds(0, JR)]], joint_v, sh).wait()
            for r in range(JR):
                is_p = (r >= m) & (r < m + T)
                for j in range(H // LANES):
                    sl = pl.ds(j * LANES, LANES)
                    joint_v[r, sl] = jnp.where(is_p, pref_v[r, sl],
                                               joint_v[r, sl])
            pltpu.sync_copy(joint_v, out_hbm.at[pl.ds(gb, JR)])

        for c in range(NCH - 1):
            cur = c % 2
            nxt = 1 - cur
            if c + 1 <= NCH - 2:
                if writes[nxt] is not None:
                    writes[nxt].wait()
                gathers[nxt] = pltpu.async_copy(
                    wte_hbm.at[idx_v.at[pl.ds(JR + (c + 1) * CH, CH)]],
                    rows_v.at[nxt], sg[nxt])
            gathers[cur].wait()
            writes[cur] = pltpu.async_copy(
                rows_v.at[cur],
                out_hbm.at[pl.ds(rb + (NCH * sub + c) * CH, CH)],
                so[cur])

        fbuf = (NCH - 1) % 2
        if writes[fbuf] is not None:
            writes[fbuf].wait()
        obase = rb + (NCH * sub + NCH - 1) * CH
        small = (b < B - 1) & (sub == WPB - 1)

        @pl.when(jnp.logical_not(small))
        def _():
            pltpu.async_copy(
                wte_hbm.at[idx_v.at[pl.ds(JR + (NCH - 1) * CH, CH)]],
                rows_v.at[fbuf], sg[fbuf]).wait()
            pltpu.sync_copy(rows_v.at[fbuf], out_hbm.at[pl.ds(obase, CH)])

        @pl.when(small)
        def _():
            pltpu.async_copy(
                wte_hbm.at[idx_v.at[pl.ds(JR + (NCH - 1) * CH, LAST)]],
                rows_v.at[fbuf, pl.ds(0, LAST)], sg[fbuf]).wait()
            pltpu.sync_copy(rows_v.at[fbuf, pl.ds(0, LAST)],
                            out_hbm.at[pl.ds(obase, LAST)])

        if writes[1 - fbuf] is not None:
            writes[1 - fbuf].wait()

    return k(wte_weight, tok_flat, pos_map, prompts)


def kernel(tokens, prefix_indices, wte_weight, input_tokens, W1, b1, W2, b2):
    B, S = tokens.shape
    P, T, MID = input_tokens.shape
    H = W1.shape[1]
    ROWS = T + S

    info = plsc.get_sparse_core_info()
    NC = info.num_cores
    NW = NC * info.num_subcores
    WPB = NW // B
    HEAD = T + (-T) % 8
    CH = 32
    NCH = 8
    IW = HEAD + NCH * CH

    G = [((ROWS * b) // 8) * 8 for b in range(B)]
    shifts = [ROWS * b - G[b] for b in range(B)]
    R = [G[b] + HEAD for b in range(B)]
    reg_len = [(G[b + 1] if b + 1 < B else B * ROWS) - R[b] for b in range(B)]
    assert reg_len[B - 1] == CH * NCH * WPB
    assert all(l == reg_len[0] for l in reg_len[:-1])
    LAST = reg_len[0] - CH * (NCH * WPB - 1)
    assert 0 < LAST <= CH and LAST % 8 == 0

    pos_map = np.zeros((NW * IW,), dtype=np.int32)
    for b in range(B):
        for sub in range(WPB):
            base = (b * WPB + sub) * IW
            if sub == 0:
                for r in range(HEAD):
                    src = _row_source(G[b] + r, S, T)
                    if src is not None:
                        pos_map[base + r] = src
            for k_ in range(NCH):
                c = NCH * sub + k_
                g0 = R[b] + CH * c
                sz = min(CH, reg_len[b] - CH * c)
                for j in range(sz):
                    pos_map[base + HEAD + CH * k_ + j] = _row_source(
                        g0 + j, S, T)

    tok_flat = tokens.astype(jnp.int32).reshape(B * S)
    prompts = _prefix_prompts(prefix_indices.astype(jnp.int32),
                              input_tokens, W1, b1, W2, b2, HEAD, shifts)
    out = _sc_gather(wte_weight, tok_flat, jnp.asarray(pos_map), prompts,
                     B, S, T, H, NC, NW, WPB, HEAD, IW, CH, NCH, LAST, ROWS)
    return out.reshape(B, ROWS, H)


# speedup vs baseline: 1.3633x; 1.3633x over previous
import functools

import jax
import jax.numpy as jnp
import numpy as np
from jax import lax
from jax.experimental import pallas as pl
from jax.experimental.pallas import tpu as pltpu
from jax.experimental.pallas import tpu_sc as plsc


def _mlp_select_body(P, T, HEAD, pidx_ref, x_ref, w1_ref, b1_ref,
                     w2_ref, b2_ref, out_ref):
    x = x_ref[...]
    h = jnp.tanh(
        jax.lax.dot(x, w1_ref[...], precision=jax.lax.Precision.HIGHEST)
        + b1_ref[...])
    y = (jax.lax.dot(h, w2_ref[...], precision=jax.lax.Precision.HIGHEST)
         + b2_ref[...])
    out_ref[...] = jnp.zeros(out_ref.shape, out_ref.dtype)
    slices = [lax.slice(y, (q * T, 0), ((q + 1) * T, y.shape[1]))
              for q in range(P)]
    B = out_ref.shape[0] // HEAD
    for b in range(B):
        p = pidx_ref[b]
        sel = slices[0]
        for q in range(1, P):
            sel = jnp.where(p == q, slices[q], sel)
        out_ref[pl.ds(b * HEAD, T), :] = sel


def _prefix_prompts(prefix_indices, input_tokens, W1, b1, W2, b2, HEAD):
    P, T, MID = input_tokens.shape
    H = W1.shape[1]
    B = prefix_indices.shape[0]
    x = input_tokens.reshape(P * T, MID)
    return pl.pallas_call(
        functools.partial(_mlp_select_body, P, T, HEAD),
        out_shape=jax.ShapeDtypeStruct((B * HEAD, H), jnp.float32),
        in_specs=[
            pl.BlockSpec(memory_space=pltpu.SMEM),
            pl.BlockSpec(memory_space=pltpu.VMEM),
            pl.BlockSpec(memory_space=pltpu.VMEM),
            pl.BlockSpec(memory_space=pltpu.VMEM),
            pl.BlockSpec(memory_space=pltpu.VMEM),
            pl.BlockSpec(memory_space=pltpu.VMEM),
        ],
        out_specs=pl.BlockSpec(memory_space=pltpu.VMEM),
    )(prefix_indices, x, W1, b1.reshape(1, H), W2, b2.reshape(1, H))


def _sc_gather(wte_weight, tok_flat, pos_map, prompts,
               B, S, T, H, NC, NW, WPB, HEAD, IW, CH, NCH, PR):
    LANES = 16
    JR = HEAD

    mesh = plsc.VectorSubcoreMesh(core_axis_name="c", subcore_axis_name="s")

    @functools.partial(
        pl.kernel,
        out_type=jax.ShapeDtypeStruct((B * PR, H), jnp.float32),
        mesh=mesh,
        scratch_types=[
            pltpu.VMEM((IW,), jnp.int32),
            pltpu.VMEM((IW,), jnp.int32),
            pltpu.VMEM((2, CH, H), jnp.float32),
            pltpu.VMEM((JR, H), jnp.float32),
            pltpu.VMEM((JR, H), jnp.float32),
            pltpu.SemaphoreType.DMA,
            pltpu.SemaphoreType.DMA,
            pltpu.SemaphoreType.DMA,
            pltpu.SemaphoreType.DMA,
            pltpu.SemaphoreType.DMA,
        ],
    )
    def k(wte_hbm, tok_hbm, map_hbm, prompts_hbm, out_hbm,
          map_v, idx_v, rows_v, pref_v, joint_v, sg0, sg1, so0, so1, sh):
        wid = lax.axis_index("c") * (NW // NC) + lax.axis_index("s")
        b = wid // WPB
        sub = wid % WPB
        pbase = b * PR

        pltpu.sync_copy(map_hbm.at[pl.ds(wid * IW, IW)], map_v)
        e0 = pltpu.async_copy(
            tok_hbm.at[map_v.at[pl.ds(0, 96)]], idx_v.at[pl.ds(0, 96)], sg0)
        e1 = pltpu.async_copy(
            tok_hbm.at[map_v.at[pl.ds(96, 96)]], idx_v.at[pl.ds(96, 96)],
            sg1)
        e2 = pltpu.async_copy(
            tok_hbm.at[map_v.at[pl.ds(192, IW - 192)]],
            idx_v.at[pl.ds(192, IW - 192)], so0)
        e0.wait()
        e1.wait()
        e2.wait()

        sg = (sg0, sg1)
        so = (so0, so1)
        gathers = [None, None]
        writes = [None, None]
        gathers[0] = pltpu.async_copy(
            wte_hbm.at[idx_v.at[pl.ds(JR, CH)]], rows_v.at[0], sg[0])

        @pl.when(sub == 0)
        def _():
            pltpu.sync_copy(prompts_hbm.at[pl.ds(b * HEAD, HEAD)], pref_v)
            pltpu.async_copy(
                wte_hbm.at[idx_v.at[pl.ds(0, JR)]], joint_v, sh).wait()
            for r in range(T, JR):
                for j in range(H // LANES):
                    sl = pl.ds(j * LANES, LANES)
                    pref_v[r, sl] = joint_v[r, sl]
            pltpu.sync_copy(pref_v, out_hbm.at[pl.ds(pbase, JR)])

        for c in range(NCH):
            cur = c % 2
            nxt = 1 - cur
            if c + 1 <= NCH - 1:
                if writes[nxt] is not None:
                    writes[nxt].wait()
                gathers[nxt] = pltpu.async_copy(
                    wte_hbm.at[idx_v.at[pl.ds(JR + (c + 1) * CH, CH)]],
                    rows_v.at[nxt], sg[nxt])
            gathers[cur].wait()
            writes[cur] = pltpu.async_copy(
                rows_v.at[cur],
                out_hbm.at[pl.ds(pbase + JR + (NCH * sub + c) * CH, CH)],
                so[cur])

        writes[NCH % 2].wait()
        writes[1 - NCH % 2].wait()

    return k(wte_weight, tok_flat, pos_map, prompts)


def kernel(tokens, prefix_indices, wte_weight, input_tokens, W1, b1, W2, b2):
    B, S = tokens.shape
    P, T, MID = input_tokens.shape
    H = W1.shape[1]
    ROWS = T + S
    PR = ROWS + (-ROWS) % 8

    info = plsc.get_sparse_core_info()
    NC = info.num_cores
    NW = NC * info.num_subcores
    WPB = NW // B
    HEAD = T + (-T) % 8
    CH = 32
    NCH = (PR - HEAD) // (WPB * CH)
    assert HEAD + WPB * NCH * CH == PR
    IW = HEAD + NCH * CH

    pos_map = np.zeros((NW * IW,), dtype=np.int32)
    for b in range(B):
        for sub in range(WPB):
            base = (b * WPB + sub) * IW
            if sub == 0:
                for r in range(T, HEAD):
                    pos_map[base + r] = b * S + (r - T)
            for k_ in range(NCH):
                c = NCH * sub + k_
                for j in range(CH):
                    r = HEAD + CH * c + j
                    pos_map[base + HEAD + CH * k_ + j] = (
                        b * S + (r - T if r < ROWS else r - HEAD))

    tok_flat = tokens.astype(jnp.int32).reshape(B * S)
    prompts = _prefix_prompts(prefix_indices.astype(jnp.int32),
                              input_tokens, W1, b1, W2, b2, HEAD)
    out = _sc_gather(wte_weight, tok_flat, jnp.asarray(pos_map), prompts,
                     B, S, T, H, NC, NW, WPB, HEAD, IW, CH, NCH, PR)
    return out.reshape(B, PR, H)[:, :ROWS, :]


# speedup vs baseline: 1.3999x; 1.0269x over previous
import functools

import jax
import jax.numpy as jnp
import numpy as np
from jax import lax
from jax.experimental import pallas as pl
from jax.experimental.pallas import tpu as pltpu
from jax.experimental.pallas import tpu_sc as plsc


def _mlp_select_body(P, T, HEAD, pidx_ref, x_ref, w1_ref, b1_ref,
                     w2_ref, b2_ref, out_ref):
    x = x_ref[...]
    h = jnp.tanh(
        jax.lax.dot(x, w1_ref[...], precision=jax.lax.Precision.HIGHEST)
        + b1_ref[...])
    y = (jax.lax.dot(h, w2_ref[...], precision=jax.lax.Precision.HIGHEST)
         + b2_ref[...])
    out_ref[...] = jnp.zeros(out_ref.shape, out_ref.dtype)
    slices = [lax.slice(y, (q * T, 0), ((q + 1) * T, y.shape[1]))
              for q in range(P)]
    B = out_ref.shape[0] // HEAD
    for b in range(B):
        p = pidx_ref[b]
        sel = slices[0]
        for q in range(1, P):
            sel = jnp.where(p == q, slices[q], sel)
        out_ref[pl.ds(b * HEAD, T), :] = sel


def _prefix_prompts(prefix_indices, input_tokens, W1, b1, W2, b2, HEAD):
    P, T, MID = input_tokens.shape
    H = W1.shape[1]
    B = prefix_indices.shape[0]
    x = input_tokens.reshape(P * T, MID)
    return pl.pallas_call(
        functools.partial(_mlp_select_body, P, T, HEAD),
        out_shape=jax.ShapeDtypeStruct((B * HEAD, H), jnp.float32),
        in_specs=[
            pl.BlockSpec(memory_space=pltpu.SMEM),
            pl.BlockSpec(memory_space=pltpu.VMEM),
            pl.BlockSpec(memory_space=pltpu.VMEM),
            pl.BlockSpec(memory_space=pltpu.VMEM),
            pl.BlockSpec(memory_space=pltpu.VMEM),
            pl.BlockSpec(memory_space=pltpu.VMEM),
        ],
        out_specs=pl.BlockSpec(memory_space=pltpu.VMEM),
    )(prefix_indices, x, W1, b1.reshape(1, H), W2, b2.reshape(1, H))


def _sc_gather(wte_weight, tok_flat, pos_map, prompts,
               B, S, T, H, NC, NW, WPB, HEAD, IW, CH, NCH, PR):
    LANES = 16
    JR = HEAD

    mesh = plsc.VectorSubcoreMesh(core_axis_name="c", subcore_axis_name="s")

    @functools.partial(
        pl.kernel,
        out_type=jax.ShapeDtypeStruct((B * PR, H), jnp.float32),
        mesh=mesh,
        scratch_types=[
            pltpu.VMEM((IW,), jnp.int32),
            pltpu.VMEM((IW,), jnp.int32),
            pltpu.VMEM((2, CH, H), jnp.float32),
            pltpu.VMEM((JR, H), jnp.float32),
            pltpu.VMEM((JR, H), jnp.float32),
            pltpu.SemaphoreType.DMA,
            pltpu.SemaphoreType.DMA,
            pltpu.SemaphoreType.DMA,
            pltpu.SemaphoreType.DMA,
            pltpu.SemaphoreType.DMA,
            pltpu.SemaphoreType.DMA,
        ],
    )
    def k(wte_hbm, tok_hbm, map_hbm, prompts_hbm, out_hbm,
          map_v, idx_v, rows_v, pref_v, joint_v, sg0, sg1, so0, so1, sh,
          sp):
        wid = lax.axis_index("c") * (NW // NC) + lax.axis_index("s")
        b = wid // WPB
        sub = wid % WPB
        pbase = b * PR

        pltpu.sync_copy(map_hbm.at[pl.ds(wid * IW, IW)], map_v)
        e0 = pltpu.async_copy(
            tok_hbm.at[map_v.at[pl.ds(0, 96)]], idx_v.at[pl.ds(0, 96)], sg0)
        e1 = pltpu.async_copy(
            tok_hbm.at[map_v.at[pl.ds(96, 96)]], idx_v.at[pl.ds(96, 96)],
            sg1)
        e2 = pltpu.async_copy(
            tok_hbm.at[map_v.at[pl.ds(192, IW - 192)]],
            idx_v.at[pl.ds(192, IW - 192)], so0)
        e0.wait()

        sg = (sg0, sg1)
        so = (so0, so1)
        gathers = [None, None]
        writes = [None, None]
        gathers[0] = pltpu.async_copy(
            wte_hbm.at[idx_v.at[pl.ds(JR, CH)]], rows_v.at[0], sg[0])

        @pl.when(sub == 0)
        def _():
            pltpu.async_copy(prompts_hbm.at[pl.ds(b * HEAD, HEAD)], pref_v,
                             sp)
            pltpu.async_copy(wte_hbm.at[idx_v.at[pl.ds(0, JR)]], joint_v,
                             sh)

        e1.wait()

        for c in range(NCH):
            cur = c % 2
            nxt = 1 - cur
            if c == 4:
                e2.wait()
            if c + 1 <= NCH - 1:
                if writes[nxt] is not None:
                    writes[nxt].wait()
                gathers[nxt] = pltpu.async_copy(
                    wte_hbm.at[idx_v.at[pl.ds(JR + (c + 1) * CH, CH)]],
                    rows_v.at[nxt], sg[nxt])
            gathers[cur].wait()
            writes[cur] = pltpu.async_copy(
                rows_v.at[cur],
                out_hbm.at[pl.ds(pbase + JR + (NCH * sub + c) * CH, CH)],
                so[cur])

        @pl.when(sub == 0)
        def _():
            pltpu.make_async_copy(prompts_hbm.at[pl.ds(b * HEAD, HEAD)],
                                  pref_v, sp).wait()
            pltpu.make_async_copy(wte_hbm.at[idx_v.at[pl.ds(0, JR)]],
                                  joint_v, sh).wait()
            for r in range(T, JR):
                for j in range(H // LANES):
                    sl = pl.ds(j * LANES, LANES)
                    pref_v[r, sl] = joint_v[r, sl]
            pltpu.sync_copy(pref_v, out_hbm.at[pl.ds(pbase, JR)])

        writes[NCH % 2].wait()
        writes[1 - NCH % 2].wait()

    return k(wte_weight, tok_flat, pos_map, prompts)


def kernel(tokens, prefix_indices, wte_weight, input_tokens, W1, b1, W2, b2):
    B, S = tokens.shape
    P, T, MID = input_tokens.shape
    H = W1.shape[1]
    ROWS = T + S
    PR = ROWS + (-ROWS) % 8

    info = plsc.get_sparse_core_info()
    NC = info.num_cores
    NW = NC * info.num_subcores
    WPB = NW // B
    HEAD = T + (-T) % 8
    CH = 32
    NCH = (PR - HEAD) // (WPB * CH)
    assert HEAD + WPB * NCH * CH == PR
    IW = HEAD + NCH * CH

    pos_map = np.zeros((NW * IW,), dtype=np.int32)
    for b in range(B):
        for sub in range(WPB):
            base = (b * WPB + sub) * IW
            if sub == 0:
                for r in range(T, HEAD):
                    pos_map[base + r] = b * S + (r - T)
            for k_ in range(NCH):
                c = NCH * sub + k_
                for j in range(CH):
                    r = HEAD + CH * c + j
                    pos_map[base + HEAD + CH * k_ + j] = (
                        b * S + (r - T if r < ROWS else r - HEAD))

    tok_flat = tokens.astype(jnp.int32).reshape(B * S)
    prompts = _prefix_prompts(prefix_indices.astype(jnp.int32),
                              input_tokens, W1, b1, W2, b2, HEAD)
    out = _sc_gather(wte_weight, tok_flat, jnp.asarray(pos_map), prompts,
                     B, S, T, H, NC, NW, WPB, HEAD, IW, CH, NCH, PR)
    return out.reshape(B, PR, H)[:, :ROWS, :]


# speedup vs baseline: 1.4682x; 1.0487x over previous
import functools

import jax
import jax.numpy as jnp
import numpy as np
from jax import lax
from jax.experimental import pallas as pl
from jax.experimental.pallas import tpu as pltpu
from jax.experimental.pallas import tpu_sc as plsc


def _mlp_select_body(P, T, HEAD, pidx_ref, x_ref, w1_ref, b1_ref,
                     w2_ref, b2_ref, out_ref):
    x = x_ref[...]
    h = jnp.tanh(
        jax.lax.dot(x, w1_ref[...], precision=jax.lax.Precision.HIGHEST)
        + b1_ref[...])
    y = (jax.lax.dot(h, w2_ref[...], precision=jax.lax.Precision.HIGHEST)
         + b2_ref[...])
    out_ref[...] = jnp.zeros(out_ref.shape, out_ref.dtype)
    slices = [lax.slice(y, (q * T, 0), ((q + 1) * T, y.shape[1]))
              for q in range(P)]
    B = out_ref.shape[0] // HEAD
    for b in range(B):
        p = pidx_ref[b]
        sel = slices[0]
        for q in range(1, P):
            sel = jnp.where(p == q, slices[q], sel)
        out_ref[pl.ds(b * HEAD, T), :] = sel


def _prefix_prompts(prefix_indices, input_tokens, W1, b1, W2, b2, HEAD):
    P, T, MID = input_tokens.shape
    H = W1.shape[1]
    B = prefix_indices.shape[0]
    x = input_tokens.reshape(P * T, MID)
    return pl.pallas_call(
        functools.partial(_mlp_select_body, P, T, HEAD),
        out_shape=jax.ShapeDtypeStruct((B * HEAD, H), jnp.float32),
        in_specs=[
            pl.BlockSpec(memory_space=pltpu.SMEM),
            pl.BlockSpec(memory_space=pltpu.VMEM),
            pl.BlockSpec(memory_space=pltpu.VMEM),
            pl.BlockSpec(memory_space=pltpu.VMEM),
            pl.BlockSpec(memory_space=pltpu.VMEM),
            pl.BlockSpec(memory_space=pltpu.VMEM),
        ],
        out_specs=pl.BlockSpec(memory_space=pltpu.VMEM),
    )(prefix_indices, x, W1, b1.reshape(1, H), W2, b2.reshape(1, H))


def _sc_gather(wte_weight, tok_flat, pos_map,
               B, S, T, H, NC, NW, WPB, HEAD, IW, CH, NCH, PR):
    JR = HEAD

    mesh = plsc.VectorSubcoreMesh(core_axis_name="c", subcore_axis_name="s")

    @functools.partial(
        pl.kernel,
        out_type=jax.ShapeDtypeStruct((B * PR, H), jnp.float32),
        mesh=mesh,
        scratch_types=[
            pltpu.VMEM((IW,), jnp.int32),
            pltpu.VMEM((IW,), jnp.int32),
            pltpu.VMEM((2, CH, H), jnp.float32),
            pltpu.VMEM((JR, H), jnp.float32),
            pltpu.SemaphoreType.DMA,
            pltpu.SemaphoreType.DMA,
            pltpu.SemaphoreType.DMA,
            pltpu.SemaphoreType.DMA,
            pltpu.SemaphoreType.DMA,
        ],
    )
    def k(wte_hbm, tok_hbm, map_hbm, out_hbm,
          map_v, idx_v, rows_v, joint_v, sg0, sg1, so0, so1, sh):
        wid = lax.axis_index("c") * (NW // NC) + lax.axis_index("s")
        b = wid // WPB
        sub = wid % WPB
        pbase = b * PR

        pltpu.sync_copy(map_hbm.at[pl.ds(wid * IW, IW)], map_v)
        e0 = pltpu.async_copy(
            tok_hbm.at[map_v.at[pl.ds(0, 96)]], idx_v.at[pl.ds(0, 96)], sg0)
        e1 = pltpu.async_copy(
            tok_hbm.at[map_v.at[pl.ds(96, 96)]], idx_v.at[pl.ds(96, 96)],
            sg1)
        e2 = pltpu.async_copy(
            tok_hbm.at[map_v.at[pl.ds(192, IW - 192)]],
            idx_v.at[pl.ds(192, IW - 192)], so0)
        e0.wait()

        sg = (sg0, sg1)
        so = (so0, so1)
        gathers = [None, None]
        writes = [None, None]
        gathers[0] = pltpu.async_copy(
            wte_hbm.at[idx_v.at[pl.ds(JR, CH)]], rows_v.at[0], sg[0])

        @pl.when(sub == 0)
        def _():
            pltpu.async_copy(wte_hbm.at[idx_v.at[pl.ds(0, JR)]], joint_v,
                             sh)

        e1.wait()

        for c in range(NCH):
            cur = c % 2
            nxt = 1 - cur
            if c == 4:
                e2.wait()
            if c + 1 <= NCH - 1:
                if writes[nxt] is not None:
                    writes[nxt].wait()
                gathers[nxt] = pltpu.async_copy(
                    wte_hbm.at[idx_v.at[pl.ds(JR + (c + 1) * CH, CH)]],
                    rows_v.at[nxt], sg[nxt])
            gathers[cur].wait()
            writes[cur] = pltpu.async_copy(
                rows_v.at[cur],
                out_hbm.at[pl.ds(pbase + JR + (NCH * sub + c) * CH, CH)],
                so[cur])

        @pl.when(sub == 0)
        def _():
            pltpu.make_async_copy(wte_hbm.at[idx_v.at[pl.ds(0, JR)]],
                                  joint_v, sh).wait()
            pltpu.sync_copy(joint_v, out_hbm.at[pl.ds(pbase, JR)])

        writes[NCH % 2].wait()
        writes[1 - NCH % 2].wait()

    return k(wte_weight, tok_flat, pos_map)


def _place_body(B, T, PR, HEAD, gath_ref, prompts_ref, out_ref, blk, sem,
                sem2):
    for b in range(B):
        pltpu.make_async_copy(
            prompts_ref.at[pl.ds(b * HEAD, 8)],
            out_ref.at[pl.ds(b * PR, 8)], sem).start()
        pltpu.make_async_copy(
            out_ref.at[pl.ds(b * PR + 8, 8)], blk.at[b], sem2).start()
    for b in range(B):
        pltpu.make_async_copy(
            out_ref.at[pl.ds(b * PR + 8, 8)], blk.at[b], sem2).wait()
        blk[b, pl.ds(0, 2), :] = prompts_ref[pl.ds(b * HEAD + 8, 2), :]
    for b in range(B):
        pltpu.make_async_copy(
            blk.at[b], out_ref.at[pl.ds(b * PR + 8, 8)], sem2).start()
    for b in range(B):
        pltpu.make_async_copy(
            prompts_ref.at[pl.ds(b * HEAD, 8)],
            out_ref.at[pl.ds(b * PR, 8)], sem).wait()
        pltpu.make_async_copy(
            blk.at[b], out_ref.at[pl.ds(b * PR + 8, 8)], sem2).wait()


def _place_prompts(gathered, prompts, B, T, PR, HEAD, H):
    return pl.pallas_call(
        functools.partial(_place_body, B, T, PR, HEAD),
        out_shape=jax.ShapeDtypeStruct(gathered.shape, gathered.dtype),
        in_specs=[
            pl.BlockSpec(memory_space=pl.ANY),
            pl.BlockSpec(memory_space=pltpu.VMEM),
        ],
        out_specs=pl.BlockSpec(memory_space=pl.ANY),
        scratch_shapes=[pltpu.VMEM((B, 8, H), jnp.float32),
                        pltpu.SemaphoreType.DMA,
                        pltpu.SemaphoreType.DMA],
        input_output_aliases={0: 0},
    )(gathered, prompts)


def kernel(tokens, prefix_indices, wte_weight, input_tokens, W1, b1, W2, b2):
    B, S = tokens.shape
    P, T, MID = input_tokens.shape
    H = W1.shape[1]
    ROWS = T + S
    PR = ROWS + (-ROWS) % 8

    info = plsc.get_sparse_core_info()
    NC = info.num_cores
    NW = NC * info.num_subcores
    WPB = NW // B
    HEAD = T + (-T) % 8
    CH = 32
    NCH = (PR - HEAD) // (WPB * CH)
    assert HEAD + WPB * NCH * CH == PR
    IW = HEAD + NCH * CH

    pos_map = np.zeros((NW * IW,), dtype=np.int32)
    for b in range(B):
        for sub in range(WPB):
            base = (b * WPB + sub) * IW
            if sub == 0:
                for r in range(T, HEAD):
                    pos_map[base + r] = b * S + (r - T)
            for k_ in range(NCH):
                c = NCH * sub + k_
                for j in range(CH):
                    r = HEAD + CH * c + j
                    pos_map[base + HEAD + CH * k_ + j] = (
                        b * S + (r - T if r < ROWS else r - HEAD))

    tok_flat = tokens.astype(jnp.int32).reshape(B * S)
    prompts = _prefix_prompts(prefix_indices.astype(jnp.int32),
                              input_tokens, W1, b1, W2, b2, HEAD)
    gathered = _sc_gather(wte_weight, tok_flat, jnp.asarray(pos_map),
                          B, S, T, H, NC, NW, WPB, HEAD, IW, CH, NCH, PR)
    out = _place_prompts(gathered, prompts, B, T, PR, HEAD, H)
    return out.reshape(B, PR, H)[:, :ROWS, :]
